# baseline (speedup 1.0000x reference)
import functools

import jax
import jax.numpy as jnp
from jax import lax
from jax.experimental import pallas as pl
from jax.experimental.pallas import tpu as pltpu
from jax.experimental.pallas import tpu_sc as plsc

_T = 2048
_H = 1024
_E = 16
_I = 512
_NG = 2
_GS = _E // _NG
_RSF = 2.5
_NEG = -1e30

_BT = 128
_NBR = 48
_MR = _NBR * _BT
_NSB = _T // _BT
_NB = _NBR + _NSB
_YR = _MR + _T


_RC = 256
_PC = 128


def _route_chunk(x, gate_w, e_bias):
    logits = jnp.dot(x, gate_w.T, preferred_element_type=jnp.float32)
    s = jax.nn.sigmoid(logits)
    sb = s + e_bias
    lane = lax.broadcasted_iota(jnp.int32, logits.shape, 1)
    gid = lane // _GS

    def top2(v):
        m1 = jnp.max(v, axis=1, keepdims=True)
        i1 = jnp.min(jnp.where(v == m1, lane, _E + 1), axis=1, keepdims=True)
        v2 = jnp.where(lane == i1, _NEG, v)
        m2 = jnp.max(v2, axis=1, keepdims=True)
        i2 = jnp.min(jnp.where(v2 == m2, lane, _E + 1), axis=1, keepdims=True)
        return m1, i1, m2, i2

    m1a, _, m2a, _ = top2(jnp.where(gid == 0, sb, _NEG))
    m1b, _, m2b, _ = top2(jnp.where(gid == 1, sb, _NEG))
    chosen = jnp.where(m1a + m2a >= m1b + m2b, 0, 1)
    masked = jnp.where(gid == chosen, sb, _NEG)
    _, e1, _, e2 = top2(masked)
    w1 = jnp.sum(jnp.where(lane == e1, s, 0.0), axis=1, keepdims=True)
    w2 = jnp.sum(jnp.where(lane == e2, s, 0.0), axis=1, keepdims=True)
    wn = w1 + w2 + 1e-20
    member = ((lane == e1) | (lane == e2)).astype(jnp.float32)
    return e1, e2, (w1 / wn) * _RSF, (w2 / wn) * _RSF, member


def _router_body(x_ref, gate_ref, bias_ref,
                 d0_ref, d1_ref, w0_ref, w1_ref, be_ref, m_ref):
    def step_a(i, carry):
        sl = pl.ds(i * _RC, _RC)
        e1, e2, w1, w2, member = _route_chunk(x_ref[sl, :], gate_ref[...],
                                              bias_ref[...])
        d0_ref[sl] = e1
        d1_ref[sl] = e2
        w0_ref[sl] = w1
        w1_ref[sl] = w2
        m_ref[sl, :] = member
        return carry

    lax.fori_loop(0, _T // _RC, step_a, 0)

    rr = lax.broadcasted_iota(jnp.int32, (_PC, _PC), 0)
    cc = lax.broadcasted_iota(jnp.int32, (_PC, _PC), 1)
    tril = (rr > cc).astype(jnp.float32)

    def step_b(i, carry):
        sl = pl.ds(i * _PC, _PC)
        mc = m_ref[sl, :]
        pos = jnp.dot(tril, mc, preferred_element_type=jnp.float32) + carry
        m_ref[sl, :] = pos
        return carry + jnp.sum(mc, axis=0, keepdims=True)

    counts = lax.fori_loop(0, _T // _PC, step_b,
                           jnp.zeros((1, _E), jnp.float32))

    pc = jnp.ceil(counts / _BT) * _BT
    uu = (lax.broadcasted_iota(jnp.int32, (_E, _E), 0)
          < lax.broadcasted_iota(jnp.int32, (_E, _E), 1)).astype(jnp.float32)
    offs = jnp.dot(pc, uu, preferred_element_type=jnp.float32)
    ends = offs + pc
    bstart = (lax.broadcasted_iota(jnp.int32, (_NB, _E), 0) * _BT
              ).astype(jnp.float32)
    be = jnp.sum((ends <= bstart).astype(jnp.int32), axis=1, keepdims=True)
    be_ref[...] = be

    lane = lax.broadcasted_iota(jnp.int32, (_RC, _E), 1)

    def step_d(i, carry):
        sl = pl.ds(i * _RC, _RC)
        slots = offs + m_ref[sl, :]
        d0_ref[sl] = jnp.sum(jnp.where(lane == d0_ref[sl], slots, 0.0),
                             axis=1, keepdims=True).astype(jnp.int32)
        d1_ref[sl] = jnp.sum(jnp.where(lane == d1_ref[sl], slots, 0.0),
                             axis=1, keepdims=True).astype(jnp.int32)
        return carry

    lax.fori_loop(0, _T // _RC, step_d, 0)


def _run_router(x, gate_w, e_bias):
    grid_spec = pltpu.PrefetchScalarGridSpec(
        num_scalar_prefetch=0,
        grid=(1,),
        in_specs=[
            pl.BlockSpec((_T, _H), lambda i: (0, 0)),
            pl.BlockSpec((_E, _H), lambda i: (0, 0)),
            pl.BlockSpec((1, _E), lambda i: (0, 0)),
        ],
        out_specs=[
            pl.BlockSpec((_T, 1), lambda i: (0, 0)),
            pl.BlockSpec((_T, 1), lambda i: (0, 0)),
            pl.BlockSpec((_T, 1), lambda i: (0, 0)),
            pl.BlockSpec((_T, 1), lambda i: (0, 0)),
            pl.BlockSpec((_NB, 1), lambda i: (0, 0)),
        ],
        scratch_shapes=[pltpu.VMEM((_T, _E), jnp.float32)],
    )
    return pl.pallas_call(
        _router_body,
        grid_spec=grid_spec,
        out_shape=[
            jax.ShapeDtypeStruct((_T, 1), jnp.int32),
            jax.ShapeDtypeStruct((_T, 1), jnp.int32),
            jax.ShapeDtypeStruct((_T, 1), jnp.float32),
            jax.ShapeDtypeStruct((_T, 1), jnp.float32),
            jax.ShapeDtypeStruct((_NB, 1), jnp.int32),
        ],
        compiler_params=pltpu.CompilerParams(
            dimension_semantics=("arbitrary",)),
    )(x, gate_w, e_bias.reshape(1, _E))


@functools.cache
def _make_sc_scatter():
    mesh = plsc.VectorSubcoreMesh(core_axis_name="c", subcore_axis_name="s")
    return functools.partial(
        pl.kernel,
        mesh=mesh,
        out_type=[jax.ShapeDtypeStruct((_MR,), jnp.int32),
                  jax.ShapeDtypeStruct((_MR,), jnp.float32)],
        scratch_types=[
            pltpu.VMEM((_MR,), jnp.int32),
            pltpu.VMEM((_MR,), jnp.float32),
            pltpu.VMEM((_T,), jnp.int32),
            pltpu.VMEM((_T,), jnp.int32),
            pltpu.VMEM((_T,), jnp.float32),
            pltpu.VMEM((_T,), jnp.float32),
        ],
        compiler_params=pltpu.CompilerParams(needs_layout_passes=False),
    )(_sc_scatter_body)


def _sc_scatter_body(d0_hbm, d1_hbm, w0_hbm, w1_hbm, src_hbm, sw_hbm,
                     st_v, swt_v, d0_v, d1_v, w0_v, w1_v):
    wid = lax.axis_index("s") * 2 + lax.axis_index("c")

    @pl.when(wid == 0)
    def _():
        zeros_i = jnp.zeros((16,), jnp.int32)
        ones_f = jnp.full((16,), 1.0, jnp.float32)

        def init(i, c):
            sl = pl.ds(i * 16, 16)
            st_v[sl] = zeros_i
            swt_v[sl] = ones_f
            return c

        lax.fori_loop(0, _MR // 16, init, 0)
        pltpu.sync_copy(d0_hbm, d0_v)
        pltpu.sync_copy(d1_hbm, d1_v)
        pltpu.sync_copy(w0_hbm, w0_v)
        pltpu.sync_copy(w1_hbm, w1_v)

        def scat(i, c):
            sl = pl.ds(i * 16, 16)
            toks = lax.iota(jnp.int32, 16) + i * 16
            i0 = d0_v[sl]
            i1 = d1_v[sl]
            plsc.store_scatter(st_v, [i0], toks)
            plsc.store_scatter(swt_v, [i0], w0_v[sl])
            plsc.store_scatter(st_v, [i1], toks)
            plsc.store_scatter(swt_v, [i1], w1_v[sl])
            return c

        lax.fori_loop(0, _T // 16, scat, 0)
        pltpu.sync_copy(st_v, src_hbm)
        pltpu.sync_copy(swt_v, sw_hbm)


_G_PER_W = _MR // 32
_GC = 16
_GNC = _G_PER_W // _GC
_GNS = 6


@functools.cache
def _make_sc_gather():
    mesh = plsc.VectorSubcoreMesh(core_axis_name="c", subcore_axis_name="s")
    return functools.partial(
        pl.kernel,
        mesh=mesh,
        out_type=jax.ShapeDtypeStruct((_MR, _H), jnp.float32),
        scratch_types=[
            [pltpu.VMEM((_GC,), jnp.int32) for _ in range(_GNC)],
            [pltpu.VMEM((_GC, _H), jnp.float32) for _ in range(_GNS)],
            pltpu.SemaphoreType.DMA,
            [pltpu.SemaphoreType.DMA for _ in range(_GNS)],
            [pltpu.SemaphoreType.DMA for _ in range(_GNS)],
        ],
        compiler_params=pltpu.CompilerParams(needs_layout_passes=False),
    )(_sc_gather_body)


def _sc_gather_body(x_hbm, src_hbm, xg_hbm, idx_vs, rows_vs, semi, semg, semw):
    wid = lax.axis_index("s") * 2 + lax.axis_index("c")
    base = wid * _G_PER_W

    cps = [pltpu.async_copy(src_hbm.at[pl.ds(base + c * _GC, _GC)],
                            idx_vs[c], semi) for c in range(_GNC)]
    for cp in cps:
        cp.wait()

    gcp = [None] * _GNS
    wcp = [None] * _GNS
    for phase in range(_GNC // _GNS):
        for j in range(_GNS):
            c = phase * _GNS + j
            if phase > 0:
                wcp[j].wait()
            gcp[j] = pltpu.async_copy(x_hbm.at[idx_vs[c]], rows_vs[j],
                                      semg[j])
        for j in range(_GNS):
            c = phase * _GNS + j
            gcp[j].wait()
            wcp[j] = pltpu.async_copy(
                rows_vs[j], xg_hbm.at[pl.ds(base + c * _GC, _GC)], semw[j])
    for j in range(_GNS):
        wcp[j].wait()


def _gmm_body(be_ref, xg_ref, sw_ref, w13_ref, w2_ref, sw13_ref, sw2_ref,
              x_ref, y_ref):
    b = pl.program_id(0)
    be = be_ref[b]

    @pl.when(jnp.logical_and(b < _NBR, be < _E))
    def _routed():
        gu = jnp.dot(xg_ref[...], w13_ref[0].T,
                     preferred_element_type=jnp.float32)
        act = jax.nn.silu(gu[:, :_I]) * gu[:, _I:]
        act = act * sw_ref[...]
        y_ref[...] = jnp.dot(act, w2_ref[0].T,
                             preferred_element_type=jnp.float32)

    @pl.when(b >= _NBR)
    def _shared():
        sgu = jnp.dot(x_ref[...], sw13_ref[...].T,
                      preferred_element_type=jnp.float32)
        sact = jax.nn.silu(sgu[:, :_I]) * sgu[:, _I:]
        y_ref[...] = jnp.dot(sact, sw2_ref[...].T,
                             preferred_element_type=jnp.float32)


def _run_gmm(be, xg, sw, w13, w2, shared_w13, shared_w2, x):
    grid_spec = pltpu.PrefetchScalarGridSpec(
        num_scalar_prefetch=1,
        grid=(_NB,),
        in_specs=[
            pl.BlockSpec((_BT, _H), lambda b, be: (jnp.minimum(b, _NBR - 1), 0)),
            pl.BlockSpec((_BT, 1), lambda b, be: (jnp.minimum(b, _NBR - 1), 0)),
            pl.BlockSpec((1, 2 * _I, _H),
                         lambda b, be: (jnp.minimum(be[b], _E - 1), 0, 0)),
            pl.BlockSpec((1, _H, _I),
                         lambda b, be: (jnp.minimum(be[b], _E - 1), 0, 0)),
            pl.BlockSpec((2 * _I, _H), lambda b, be: (0, 0)),
            pl.BlockSpec((_H, _I), lambda b, be: (0, 0)),
            pl.BlockSpec((_BT, _H),
                         lambda b, be: (jnp.maximum(b - _NBR, 0), 0)),
        ],
        out_specs=pl.BlockSpec((_BT, _H), lambda b, be: (b, 0)),
    )
    return pl.pallas_call(
        _gmm_body,
        grid_spec=grid_spec,
        out_shape=jax.ShapeDtypeStruct((_YR, _H), jnp.float32),
        compiler_params=pltpu.CompilerParams(
            dimension_semantics=("arbitrary",)),
    )(be, xg, sw, w13, w2, shared_w13, shared_w2, x)


_C_PER_W = _T // 32
_CC = 16


_CNC = _C_PER_W // _CC


@functools.cache
def _make_sc_combine():
    mesh = plsc.VectorSubcoreMesh(core_axis_name="c", subcore_axis_name="s")
    return functools.partial(
        pl.kernel,
        mesh=mesh,
        out_type=jax.ShapeDtypeStruct((_T, _H), jnp.float32),
        scratch_types=[
            [pltpu.VMEM((_CC,), jnp.int32) for _ in range(_CNC)],
            [pltpu.VMEM((_CC,), jnp.int32) for _ in range(_CNC)],
            [pltpu.VMEM((_CC, _H), jnp.float32) for _ in range(2)],
            [pltpu.VMEM((_CC, _H), jnp.float32) for _ in range(2)],
            [pltpu.VMEM((_CC, _H), jnp.float32) for _ in range(2)],
            pltpu.SemaphoreType.DMA,
            [pltpu.SemaphoreType.DMA for _ in range(2)],
            [pltpu.SemaphoreType.DMA for _ in range(2)],
        ],
        compiler_params=pltpu.CompilerParams(needs_layout_passes=False),
    )(_sc_combine_body)


def _sc_combine_body(y_hbm, d0_hbm, d1_hbm, out_hbm,
                     i0_vs, i1_vs, b0_vs, b1_vs, bs_vs, semi, semg, semw):
    wid = lax.axis_index("s") * 2 + lax.axis_index("c")
    base = wid * _C_PER_W

    cps = [pltpu.async_copy(d0_hbm.at[pl.ds(base + c * _CC, _CC)],
                            i0_vs[c], semi) for c in range(_CNC)]
    cps += [pltpu.async_copy(d1_hbm.at[pl.ds(base + c * _CC, _CC)],
                             i1_vs[c], semi) for c in range(_CNC)]
    for cp in cps:
        cp.wait()

    gcp = [None, None]
    wcp = [None, None]

    def start_gathers(c):
        k = c % 2
        gcp[k] = [
            pltpu.async_copy(y_hbm.at[i0_vs[c]], b0_vs[k], semg[k]),
            pltpu.async_copy(y_hbm.at[i1_vs[c]], b1_vs[k], semg[k]),
            pltpu.async_copy(y_hbm.at[pl.ds(_MR + base + c * _CC, _CC)],
                             bs_vs[k], semg[k]),
        ]

    start_gathers(0)
    for c in range(_CNC):
        k = c % 2
        for cp in gcp[k]:
            cp.wait()
        if c + 1 < _CNC:
            if c >= 1:
                wcp[(c + 1) % 2].wait()
            start_gathers(c + 1)

        def row(r, cr):
            def grp(g, cg):
                sl = pl.ds(g * 16, 16)
                b0_vs[k][r, sl] = (b0_vs[k][r, sl] + b1_vs[k][r, sl]
                                   + bs_vs[k][r, sl])
                return cg
            lax.fori_loop(0, _H // 16, grp, 0)
            return cr

        lax.fori_loop(0, _CC, row, 0)
        wcp[k] = pltpu.async_copy(
            b0_vs[k], out_hbm.at[pl.ds(base + c * _CC, _CC)], semw[k])
    wcp[0].wait()
    wcp[1].wait()


@jax.jit
def kernel(hidden_states, gate_w, e_bias, w13, w2, shared_w13, shared_w2):
    d0, d1, w0, w1, be = _run_router(hidden_states, gate_w, e_bias)
    src, sw = _make_sc_scatter()(d0.reshape(_T), d1.reshape(_T),
                                 w0.reshape(_T), w1.reshape(_T))
    xg = _make_sc_gather()(hidden_states, src)
    y = _run_gmm(be.reshape(_NB), xg, sw.reshape(_MR, 1),
                 w13, w2, shared_w13, shared_w2, hidden_states)
    return _make_sc_combine()(y, d0.reshape(_T), d1.reshape(_T))


# speedup vs baseline: 1.5411x; 1.5411x over previous
import functools

import jax
import jax.numpy as jnp
from jax import lax
from jax.experimental import pallas as pl
from jax.experimental.pallas import tpu as pltpu
from jax.experimental.pallas import tpu_sc as plsc

_T = 2048
_H = 1024
_E = 16
_I = 512
_NG = 2
_GS = _E // _NG
_RSF = 2.5
_NEG = -1e30

_BT = 128
_NBR = 48
_MR = _NBR * _BT
_NSB = _T // _BT
_NB = _NBR + _NSB
_YR = _MR + _T


_RC = 256
_PC = 128


def _route_chunk(x, gate_w, e_bias):
    logits = jnp.dot(x, gate_w.T, preferred_element_type=jnp.float32)
    s = jax.nn.sigmoid(logits)
    sb = s + e_bias
    lane = lax.broadcasted_iota(jnp.int32, logits.shape, 1)
    gid = lane // _GS

    def top2(v):
        m1 = jnp.max(v, axis=1, keepdims=True)
        i1 = jnp.min(jnp.where(v == m1, lane, _E + 1), axis=1, keepdims=True)
        v2 = jnp.where(lane == i1, _NEG, v)
        m2 = jnp.max(v2, axis=1, keepdims=True)
        i2 = jnp.min(jnp.where(v2 == m2, lane, _E + 1), axis=1, keepdims=True)
        return m1, i1, m2, i2

    m1a, _, m2a, _ = top2(jnp.where(gid == 0, sb, _NEG))
    m1b, _, m2b, _ = top2(jnp.where(gid == 1, sb, _NEG))
    chosen = jnp.where(m1a + m2a >= m1b + m2b, 0, 1)
    masked = jnp.where(gid == chosen, sb, _NEG)
    _, e1, _, e2 = top2(masked)
    w1 = jnp.sum(jnp.where(lane == e1, s, 0.0), axis=1, keepdims=True)
    w2 = jnp.sum(jnp.where(lane == e2, s, 0.0), axis=1, keepdims=True)
    wn = w1 + w2 + 1e-20
    member = ((lane == e1) | (lane == e2)).astype(jnp.float32)
    return e1, e2, (w1 / wn) * _RSF, (w2 / wn) * _RSF, member


def _router_body(x_ref, gate_ref, bias_ref,
                 d0_ref, d1_ref, w0_ref, w1_ref, be_ref, m_ref):
    def step_a(i, carry):
        sl = pl.ds(i * _RC, _RC)
        e1, e2, w1, w2, member = _route_chunk(x_ref[sl, :], gate_ref[...],
                                              bias_ref[...])
        d0_ref[sl] = e1
        d1_ref[sl] = e2
        w0_ref[sl] = w1
        w1_ref[sl] = w2
        m_ref[sl, :] = member
        return carry

    lax.fori_loop(0, _T // _RC, step_a, 0)

    rr = lax.broadcasted_iota(jnp.int32, (_PC, _PC), 0)
    cc = lax.broadcasted_iota(jnp.int32, (_PC, _PC), 1)
    tril = (rr > cc).astype(jnp.float32)

    def step_b(i, carry):
        sl = pl.ds(i * _PC, _PC)
        mc = m_ref[sl, :]
        pos = jnp.dot(tril, mc, preferred_element_type=jnp.float32) + carry
        m_ref[sl, :] = pos
        return carry + jnp.sum(mc, axis=0, keepdims=True)

    counts = lax.fori_loop(0, _T // _PC, step_b,
                           jnp.zeros((1, _E), jnp.float32))

    pc = jnp.ceil(counts / _BT) * _BT
    uu = (lax.broadcasted_iota(jnp.int32, (_E, _E), 0)
          < lax.broadcasted_iota(jnp.int32, (_E, _E), 1)).astype(jnp.float32)
    offs = jnp.dot(pc, uu, preferred_element_type=jnp.float32)
    ends = offs + pc
    bstart = (lax.broadcasted_iota(jnp.int32, (_NB, _E), 0) * _BT
              ).astype(jnp.float32)
    be = jnp.sum((ends <= bstart).astype(jnp.int32), axis=1, keepdims=True)
    be_ref[...] = be

    lane = lax.broadcasted_iota(jnp.int32, (_RC, _E), 1)

    def step_d(i, carry):
        sl = pl.ds(i * _RC, _RC)
        slots = offs + m_ref[sl, :]
        d0_ref[sl] = jnp.sum(jnp.where(lane == d0_ref[sl], slots, 0.0),
                             axis=1, keepdims=True).astype(jnp.int32)
        d1_ref[sl] = jnp.sum(jnp.where(lane == d1_ref[sl], slots, 0.0),
                             axis=1, keepdims=True).astype(jnp.int32)
        return carry

    lax.fori_loop(0, _T // _RC, step_d, 0)


def _run_router(x, gate_w, e_bias):
    grid_spec = pltpu.PrefetchScalarGridSpec(
        num_scalar_prefetch=0,
        grid=(1,),
        in_specs=[
            pl.BlockSpec((_T, _H), lambda i: (0, 0)),
            pl.BlockSpec((_E, _H), lambda i: (0, 0)),
            pl.BlockSpec((1, _E), lambda i: (0, 0)),
        ],
        out_specs=[
            pl.BlockSpec((_T, 1), lambda i: (0, 0)),
            pl.BlockSpec((_T, 1), lambda i: (0, 0)),
            pl.BlockSpec((_T, 1), lambda i: (0, 0)),
            pl.BlockSpec((_T, 1), lambda i: (0, 0)),
            pl.BlockSpec((_NB, 1), lambda i: (0, 0)),
        ],
        scratch_shapes=[pltpu.VMEM((_T, _E), jnp.float32)],
    )
    return pl.pallas_call(
        _router_body,
        grid_spec=grid_spec,
        out_shape=[
            jax.ShapeDtypeStruct((_T, 1), jnp.int32),
            jax.ShapeDtypeStruct((_T, 1), jnp.int32),
            jax.ShapeDtypeStruct((_T, 1), jnp.float32),
            jax.ShapeDtypeStruct((_T, 1), jnp.float32),
            jax.ShapeDtypeStruct((_NB, 1), jnp.int32),
        ],
        compiler_params=pltpu.CompilerParams(
            dimension_semantics=("arbitrary",)),
    )(x, gate_w, e_bias.reshape(1, _E))


@functools.cache
def _make_sc_scatter():
    mesh = plsc.VectorSubcoreMesh(core_axis_name="c", subcore_axis_name="s")
    return functools.partial(
        pl.kernel,
        mesh=mesh,
        out_type=jax.ShapeDtypeStruct((_MR,), jnp.float32),
        scratch_types=[
            pltpu.VMEM((_MR,), jnp.float32),
            pltpu.VMEM((_T,), jnp.int32),
            pltpu.VMEM((_T,), jnp.int32),
            pltpu.VMEM((_T,), jnp.float32),
            pltpu.VMEM((_T,), jnp.float32),
        ],
        compiler_params=pltpu.CompilerParams(needs_layout_passes=False),
    )(_sc_scatter_body)


def _sc_scatter_body(d0_hbm, d1_hbm, w0_hbm, w1_hbm, sw_hbm,
                     swt_v, d0_v, d1_v, w0_v, w1_v):
    wid = lax.axis_index("s") * 2 + lax.axis_index("c")

    @pl.when(wid == 0)
    def _():
        ones_f = jnp.full((16,), 1.0, jnp.float32)

        def init(i, c):
            swt_v[pl.ds(i * 16, 16)] = ones_f
            return c

        lax.fori_loop(0, _MR // 16, init, 0)
        pltpu.sync_copy(d0_hbm, d0_v)
        pltpu.sync_copy(d1_hbm, d1_v)
        pltpu.sync_copy(w0_hbm, w0_v)
        pltpu.sync_copy(w1_hbm, w1_v)

        def scat(i, c):
            sl = pl.ds(i * 16, 16)
            plsc.store_scatter(swt_v, [d0_v[sl]], w0_v[sl])
            plsc.store_scatter(swt_v, [d1_v[sl]], w1_v[sl])
            return c

        lax.fori_loop(0, _T // 16, scat, 0)
        pltpu.sync_copy(swt_v, sw_hbm)


_D_PER_W = _T // 32
_DC = 16
_DNC = _D_PER_W // _DC


@functools.cache
def _make_sc_dispatch():
    mesh = plsc.VectorSubcoreMesh(core_axis_name="c", subcore_axis_name="s")
    return functools.partial(
        pl.kernel,
        mesh=mesh,
        out_type=jax.ShapeDtypeStruct((_MR, _H), jnp.float32),
        scratch_types=[
            [pltpu.VMEM((_DC,), jnp.int32) for _ in range(_DNC)],
            [pltpu.VMEM((_DC,), jnp.int32) for _ in range(_DNC)],
            [pltpu.VMEM((_DC, _H), jnp.float32) for _ in range(2)],
            pltpu.SemaphoreType.DMA,
            [pltpu.SemaphoreType.DMA for _ in range(2)],
            [pltpu.SemaphoreType.DMA for _ in range(2)],
        ],
        compiler_params=pltpu.CompilerParams(needs_layout_passes=False),
    )(_sc_dispatch_body)


def _sc_dispatch_body(x_hbm, d0_hbm, d1_hbm, xg_hbm,
                      i0_vs, i1_vs, xbufs, semi, semx, semw):
    wid = lax.axis_index("s") * 2 + lax.axis_index("c")
    base = wid * _D_PER_W

    cps = [pltpu.async_copy(d0_hbm.at[pl.ds(base + c * _DC, _DC)],
                            i0_vs[c], semi) for c in range(_DNC)]
    cps += [pltpu.async_copy(d1_hbm.at[pl.ds(base + c * _DC, _DC)],
                             i1_vs[c], semi) for c in range(_DNC)]
    for cp in cps:
        cp.wait()

    lcp = [None, None]
    wcp = [None, None]

    def start_load(c):
        lcp[c % 2] = pltpu.async_copy(
            x_hbm.at[pl.ds(base + c * _DC, _DC)], xbufs[c % 2], semx[c % 2])

    start_load(0)
    for c in range(_DNC):
        k = c % 2
        lcp[k].wait()
        if c + 1 < _DNC:
            if c >= 1:
                for cp in wcp[(c + 1) % 2]:
                    cp.wait()
            start_load(c + 1)
        wcp[k] = [
            pltpu.async_copy(xbufs[k], xg_hbm.at[i0_vs[c]], semw[k]),
            pltpu.async_copy(xbufs[k], xg_hbm.at[i1_vs[c]], semw[k]),
        ]
    for cp in wcp[0] + wcp[1]:
        cp.wait()


def _gmm_body(be_ref, xg_ref, sw_ref, w13_ref, w2_ref, sw13_ref, sw2_ref,
              x_ref, y_ref):
    b = pl.program_id(0)
    be = be_ref[b]

    @pl.when(jnp.logical_and(b < _NBR, be < _E))
    def _routed():
        gu = jnp.dot(xg_ref[...], w13_ref[0].T,
                     preferred_element_type=jnp.float32)
        act = jax.nn.silu(gu[:, :_I]) * gu[:, _I:]
        act = act * sw_ref[...]
        y_ref[...] = jnp.dot(act, w2_ref[0].T,
                             preferred_element_type=jnp.float32)

    @pl.when(b >= _NBR)
    def _shared():
        sgu = jnp.dot(x_ref[...], sw13_ref[...].T,
                      preferred_element_type=jnp.float32)
        sact = jax.nn.silu(sgu[:, :_I]) * sgu[:, _I:]
        y_ref[...] = jnp.dot(sact, sw2_ref[...].T,
                             preferred_element_type=jnp.float32)


def _run_gmm(be, xg, sw, w13, w2, shared_w13, shared_w2, x):
    grid_spec = pltpu.PrefetchScalarGridSpec(
        num_scalar_prefetch=1,
        grid=(_NB,),
        in_specs=[
            pl.BlockSpec((_BT, _H), lambda b, be: (jnp.minimum(b, _NBR - 1), 0)),
            pl.BlockSpec((_BT, 1), lambda b, be: (jnp.minimum(b, _NBR - 1), 0)),
            pl.BlockSpec((1, 2 * _I, _H),
                         lambda b, be: (jnp.minimum(be[b], _E - 1), 0, 0)),
            pl.BlockSpec((1, _H, _I),
                         lambda b, be: (jnp.minimum(be[b], _E - 1), 0, 0)),
            pl.BlockSpec((2 * _I, _H), lambda b, be: (0, 0)),
            pl.BlockSpec((_H, _I), lambda b, be: (0, 0)),
            pl.BlockSpec((_BT, _H),
                         lambda b, be: (jnp.maximum(b - _NBR, 0), 0)),
        ],
        out_specs=pl.BlockSpec((_BT, _H), lambda b, be: (b, 0)),
    )
    return pl.pallas_call(
        _gmm_body,
        grid_spec=grid_spec,
        out_shape=jax.ShapeDtypeStruct((_YR, _H), jnp.float32),
        compiler_params=pltpu.CompilerParams(
            dimension_semantics=("arbitrary",)),
    )(be, xg, sw, w13, w2, shared_w13, shared_w2, x)


_C_PER_W = _T // 32
_CC = 16


_CNC = _C_PER_W // _CC


@functools.cache
def _make_sc_combine():
    mesh = plsc.VectorSubcoreMesh(core_axis_name="c", subcore_axis_name="s")
    return functools.partial(
        pl.kernel,
        mesh=mesh,
        out_type=jax.ShapeDtypeStruct((_T, _H), jnp.float32),
        scratch_types=[
            [pltpu.VMEM((_CC,), jnp.int32) for _ in range(_CNC)],
            [pltpu.VMEM((_CC,), jnp.int32) for _ in range(_CNC)],
            [pltpu.VMEM((_CC, _H), jnp.float32) for _ in range(2)],
            [pltpu.VMEM((_CC, _H), jnp.float32) for _ in range(2)],
            [pltpu.VMEM((_CC, _H), jnp.float32) for _ in range(2)],
            pltpu.SemaphoreType.DMA,
            [pltpu.SemaphoreType.DMA for _ in range(2)],
            [pltpu.SemaphoreType.DMA for _ in range(2)],
        ],
        compiler_params=pltpu.CompilerParams(needs_layout_passes=False),
    )(_sc_combine_body)


def _sc_combine_body(y_hbm, d0_hbm, d1_hbm, out_hbm,
                     i0_vs, i1_vs, b0_vs, b1_vs, bs_vs, semi, semg, semw):
    wid = lax.axis_index("s") * 2 + lax.axis_index("c")
    base = wid * _C_PER_W

    cps = [pltpu.async_copy(d0_hbm.at[pl.ds(base + c * _CC, _CC)],
                            i0_vs[c], semi) for c in range(_CNC)]
    cps += [pltpu.async_copy(d1_hbm.at[pl.ds(base + c * _CC, _CC)],
                             i1_vs[c], semi) for c in range(_CNC)]
    for cp in cps:
        cp.wait()

    gcp = [None, None]
    wcp = [None, None]

    def start_gathers(c):
        k = c % 2
        gcp[k] = [
            pltpu.async_copy(y_hbm.at[i0_vs[c]], b0_vs[k], semg[k]),
            pltpu.async_copy(y_hbm.at[i1_vs[c]], b1_vs[k], semg[k]),
            pltpu.async_copy(y_hbm.at[pl.ds(_MR + base + c * _CC, _CC)],
                             bs_vs[k], semg[k]),
        ]

    start_gathers(0)
    for c in range(_CNC):
        k = c % 2
        for cp in gcp[k]:
            cp.wait()
        if c + 1 < _CNC:
            if c >= 1:
                wcp[(c + 1) % 2].wait()
            start_gathers(c + 1)

        def row(r, cr):
            def grp(g, cg):
                sl = pl.ds(g * 16, 16)
                b0_vs[k][r, sl] = (b0_vs[k][r, sl] + b1_vs[k][r, sl]
                                   + bs_vs[k][r, sl])
                return cg
            lax.fori_loop(0, _H // 16, grp, 0)
            return cr

        lax.fori_loop(0, _CC, row, 0)
        wcp[k] = pltpu.async_copy(
            b0_vs[k], out_hbm.at[pl.ds(base + c * _CC, _CC)], semw[k])
    wcp[0].wait()
    wcp[1].wait()


@jax.jit
def kernel(hidden_states, gate_w, e_bias, w13, w2, shared_w13, shared_w2):
    d0, d1, w0, w1, be = _run_router(hidden_states, gate_w, e_bias)
    sw = _make_sc_scatter()(d0.reshape(_T), d1.reshape(_T),
                            w0.reshape(_T), w1.reshape(_T))
    xg = _make_sc_dispatch()(hidden_states, d0.reshape(_T), d1.reshape(_T))
    y = _run_gmm(be.reshape(_NB), xg, sw.reshape(_MR, 1),
                 w13, w2, shared_w13, shared_w2, hidden_states)
    return _make_sc_combine()(y, d0.reshape(_T), d1.reshape(_T))


# speedup vs baseline: 1.9835x; 1.2871x over previous
import functools

import jax
import jax.numpy as jnp
from jax import lax
from jax.experimental import pallas as pl
from jax.experimental.pallas import tpu as pltpu
from jax.experimental.pallas import tpu_sc as plsc

_T = 2048
_H = 1024
_E = 16
_I = 512
_NG = 2
_GS = _E // _NG
_RSF = 2.5
_NEG = -1e30

_BT = 256
_NBR = 32
_MR = _NBR * _BT
_NB = _NBR
_SB = 256


_RC = 256
_PC = 128


def _route_chunk(x, gate_w, e_bias):
    logits = jnp.dot(x, gate_w.T, preferred_element_type=jnp.float32)
    s = jax.nn.sigmoid(logits)
    sb = s + e_bias
    lane = lax.broadcasted_iota(jnp.int32, logits.shape, 1)
    gid = lane // _GS

    def top2(v):
        m1 = jnp.max(v, axis=1, keepdims=True)
        i1 = jnp.min(jnp.where(v == m1, lane, _E + 1), axis=1, keepdims=True)
        v2 = jnp.where(lane == i1, _NEG, v)
        m2 = jnp.max(v2, axis=1, keepdims=True)
        i2 = jnp.min(jnp.where(v2 == m2, lane, _E + 1), axis=1, keepdims=True)
        return m1, i1, m2, i2

    m1a, _, m2a, _ = top2(jnp.where(gid == 0, sb, _NEG))
    m1b, _, m2b, _ = top2(jnp.where(gid == 1, sb, _NEG))
    chosen = jnp.where(m1a + m2a >= m1b + m2b, 0, 1)
    masked = jnp.where(gid == chosen, sb, _NEG)
    _, e1, _, e2 = top2(masked)
    w1 = jnp.sum(jnp.where(lane == e1, s, 0.0), axis=1, keepdims=True)
    w2 = jnp.sum(jnp.where(lane == e2, s, 0.0), axis=1, keepdims=True)
    wn = w1 + w2 + 1e-20
    member = ((lane == e1) | (lane == e2)).astype(jnp.float32)
    return e1, e2, (w1 / wn) * _RSF, (w2 / wn) * _RSF, member


def _router_body(x_ref, gate_ref, bias_ref,
                 d0_ref, d1_ref, w0_ref, w1_ref, be_ref, m_ref):
    def step_a(i, carry):
        sl = pl.ds(i * _RC, _RC)
        e1, e2, w1, w2, member = _route_chunk(x_ref[sl, :], gate_ref[...],
                                              bias_ref[...])
        d0_ref[sl] = e1
        d1_ref[sl] = e2
        w0_ref[sl] = w1
        w1_ref[sl] = w2
        m_ref[sl, :] = member
        return carry

    lax.fori_loop(0, _T // _RC, step_a, 0)

    rr = lax.broadcasted_iota(jnp.int32, (_PC, _PC), 0)
    cc = lax.broadcasted_iota(jnp.int32, (_PC, _PC), 1)
    tril = (rr > cc).astype(jnp.float32)

    def step_b(i, carry):
        sl = pl.ds(i * _PC, _PC)
        mc = m_ref[sl, :]
        pos = jnp.dot(tril, mc, preferred_element_type=jnp.float32) + carry
        m_ref[sl, :] = pos
        return carry + jnp.sum(mc, axis=0, keepdims=True)

    counts = lax.fori_loop(0, _T // _PC, step_b,
                           jnp.zeros((1, _E), jnp.float32))

    pc = jnp.ceil(counts / _BT) * _BT
    uu = (lax.broadcasted_iota(jnp.int32, (_E, _E), 0)
          < lax.broadcasted_iota(jnp.int32, (_E, _E), 1)).astype(jnp.float32)
    offs = jnp.dot(pc, uu, preferred_element_type=jnp.float32)
    ends = offs + pc
    bstart = (lax.broadcasted_iota(jnp.int32, (_NB, _E), 0) * _BT
              ).astype(jnp.float32)
    be = jnp.sum((ends <= bstart).astype(jnp.int32), axis=1, keepdims=True)
    be_ref[...] = be

    lane = lax.broadcasted_iota(jnp.int32, (_RC, _E), 1)

    def step_d(i, carry):
        sl = pl.ds(i * _RC, _RC)
        slots = offs + m_ref[sl, :]
        d0_ref[sl] = jnp.sum(jnp.where(lane == d0_ref[sl], slots, 0.0),
                             axis=1, keepdims=True).astype(jnp.int32)
        d1_ref[sl] = jnp.sum(jnp.where(lane == d1_ref[sl], slots, 0.0),
                             axis=1, keepdims=True).astype(jnp.int32)
        return carry

    lax.fori_loop(0, _T // _RC, step_d, 0)


def _run_router(x, gate_w, e_bias):
    grid_spec = pltpu.PrefetchScalarGridSpec(
        num_scalar_prefetch=0,
        grid=(1,),
        in_specs=[
            pl.BlockSpec((_T, _H), lambda i: (0, 0)),
            pl.BlockSpec((_E, _H), lambda i: (0, 0)),
            pl.BlockSpec((1, _E), lambda i: (0, 0)),
        ],
        out_specs=[
            pl.BlockSpec((_T, 1), lambda i: (0, 0)),
            pl.BlockSpec((_T, 1), lambda i: (0, 0)),
            pl.BlockSpec((_T, 1), lambda i: (0, 0)),
            pl.BlockSpec((_T, 1), lambda i: (0, 0)),
            pl.BlockSpec((_NB, 1), lambda i: (0, 0)),
        ],
        scratch_shapes=[pltpu.VMEM((_T, _E), jnp.float32)],
    )
    return pl.pallas_call(
        _router_body,
        grid_spec=grid_spec,
        out_shape=[
            jax.ShapeDtypeStruct((_T, 1), jnp.int32),
            jax.ShapeDtypeStruct((_T, 1), jnp.int32),
            jax.ShapeDtypeStruct((_T, 1), jnp.float32),
            jax.ShapeDtypeStruct((_T, 1), jnp.float32),
            jax.ShapeDtypeStruct((_NB, 1), jnp.int32),
        ],
        compiler_params=pltpu.CompilerParams(
            dimension_semantics=("arbitrary",)),
    )(x, gate_w, e_bias.reshape(1, _E))


@functools.cache
def _make_sc_scatter():
    mesh = plsc.VectorSubcoreMesh(core_axis_name="c", subcore_axis_name="s")
    return functools.partial(
        pl.kernel,
        mesh=mesh,
        out_type=jax.ShapeDtypeStruct((_MR,), jnp.float32),
        scratch_types=[
            pltpu.VMEM((_MR,), jnp.float32),
            pltpu.VMEM((_T,), jnp.int32),
            pltpu.VMEM((_T,), jnp.int32),
            pltpu.VMEM((_T,), jnp.float32),
            pltpu.VMEM((_T,), jnp.float32),
        ],
        compiler_params=pltpu.CompilerParams(needs_layout_passes=False),
    )(_sc_scatter_body)


def _sc_scatter_body(d0_hbm, d1_hbm, w0_hbm, w1_hbm, sw_hbm,
                     swt_v, d0_v, d1_v, w0_v, w1_v):
    wid = lax.axis_index("s") * 2 + lax.axis_index("c")

    @pl.when(wid == 0)
    def _():
        ones_f = jnp.full((16,), 1.0, jnp.float32)

        def init(i, c):
            swt_v[pl.ds(i * 16, 16)] = ones_f
            return c

        lax.fori_loop(0, _MR // 16, init, 0)
        pltpu.sync_copy(d0_hbm, d0_v)
        pltpu.sync_copy(d1_hbm, d1_v)
        pltpu.sync_copy(w0_hbm, w0_v)
        pltpu.sync_copy(w1_hbm, w1_v)

        def scat(i, c):
            sl = pl.ds(i * 16, 16)
            plsc.store_scatter(swt_v, [d0_v[sl]], w0_v[sl])
            plsc.store_scatter(swt_v, [d1_v[sl]], w1_v[sl])
            return c

        lax.fori_loop(0, _T // 16, scat, 0)
        pltpu.sync_copy(swt_v, sw_hbm)


_D_PER_W = _T // 32
_DC = 16
_DNC = _D_PER_W // _DC


@functools.cache
def _make_sc_dispatch():
    mesh = plsc.VectorSubcoreMesh(core_axis_name="c", subcore_axis_name="s")
    return functools.partial(
        pl.kernel,
        mesh=mesh,
        out_type=jax.ShapeDtypeStruct((_MR, _H), jnp.float32),
        scratch_types=[
            [pltpu.VMEM((_DC,), jnp.int32) for _ in range(_DNC)],
            [pltpu.VMEM((_DC,), jnp.int32) for _ in range(_DNC)],
            [pltpu.VMEM((_DC, _H), jnp.float32) for _ in range(2)],
            pltpu.SemaphoreType.DMA,
            [pltpu.SemaphoreType.DMA for _ in range(2)],
            [pltpu.SemaphoreType.DMA for _ in range(2)],
        ],
        compiler_params=pltpu.CompilerParams(needs_layout_passes=False),
    )(_sc_dispatch_body)


def _sc_dispatch_body(x_hbm, d0_hbm, d1_hbm, xg_hbm,
                      i0_vs, i1_vs, xbufs, semi, semx, semw):
    wid = lax.axis_index("s") * 2 + lax.axis_index("c")
    base = wid * _D_PER_W

    cps = [pltpu.async_copy(d0_hbm.at[pl.ds(base + c * _DC, _DC)],
                            i0_vs[c], semi) for c in range(_DNC)]
    cps += [pltpu.async_copy(d1_hbm.at[pl.ds(base + c * _DC, _DC)],
                             i1_vs[c], semi) for c in range(_DNC)]
    for cp in cps:
        cp.wait()

    lcp = [None, None]
    wcp = [None, None]

    def start_load(c):
        lcp[c % 2] = pltpu.async_copy(
            x_hbm.at[pl.ds(base + c * _DC, _DC)], xbufs[c % 2], semx[c % 2])

    start_load(0)
    for c in range(_DNC):
        k = c % 2
        lcp[k].wait()
        if c + 1 < _DNC:
            if c >= 1:
                for cp in wcp[(c + 1) % 2]:
                    cp.wait()
            start_load(c + 1)
        wcp[k] = [
            pltpu.async_copy(xbufs[k], xg_hbm.at[i0_vs[c]], semw[k]),
            pltpu.async_copy(xbufs[k], xg_hbm.at[i1_vs[c]], semw[k]),
        ]
    for cp in wcp[0] + wcp[1]:
        cp.wait()


def _gmm_body(be_ref, xg_ref, sw_ref, w13_ref, w2_ref, y_ref):
    b = pl.program_id(0)
    be = be_ref[b]

    @pl.when(be < _E)
    def _routed():
        gu = jnp.dot(xg_ref[...], w13_ref[0].T,
                     preferred_element_type=jnp.float32)
        act = jax.nn.silu(gu[:, :_I]) * gu[:, _I:]
        act = act * sw_ref[...]
        y_ref[...] = jnp.dot(act, w2_ref[0].T,
                             preferred_element_type=jnp.float32)


def _shared_body(x_ref, sw13_ref, sw2_ref, ys_ref):
    sgu = jnp.dot(x_ref[...], sw13_ref[...].T,
                  preferred_element_type=jnp.float32)
    sact = jax.nn.silu(sgu[:, :_I]) * sgu[:, _I:]
    ys_ref[...] = jnp.dot(sact, sw2_ref[...].T,
                          preferred_element_type=jnp.float32)


def _run_shared(x, shared_w13, shared_w2):
    return pl.pallas_call(
        _shared_body,
        grid=(_T // _SB,),
        in_specs=[
            pl.BlockSpec((_SB, _H), lambda i: (i, 0)),
            pl.BlockSpec((2 * _I, _H), lambda i: (0, 0)),
            pl.BlockSpec((_H, _I), lambda i: (0, 0)),
        ],
        out_specs=pl.BlockSpec((_SB, _H), lambda i: (i, 0)),
        out_shape=jax.ShapeDtypeStruct((_T, _H), jnp.float32),
        compiler_params=pltpu.CompilerParams(
            dimension_semantics=("arbitrary",)),
    )(x, shared_w13, shared_w2)


def _run_gmm(be, xg, sw, w13, w2):
    grid_spec = pltpu.PrefetchScalarGridSpec(
        num_scalar_prefetch=1,
        grid=(_NB,),
        in_specs=[
            pl.BlockSpec((_BT, _H), lambda b, be: (b, 0)),
            pl.BlockSpec((_BT, 1), lambda b, be: (b, 0)),
            pl.BlockSpec((1, 2 * _I, _H),
                         lambda b, be: (jnp.minimum(be[b], _E - 1), 0, 0)),
            pl.BlockSpec((1, _H, _I),
                         lambda b, be: (jnp.minimum(be[b], _E - 1), 0, 0)),
        ],
        out_specs=pl.BlockSpec((_BT, _H), lambda b, be: (b, 0)),
    )
    return pl.pallas_call(
        _gmm_body,
        grid_spec=grid_spec,
        out_shape=jax.ShapeDtypeStruct((_MR, _H), jnp.float32),
        compiler_params=pltpu.CompilerParams(
            dimension_semantics=("arbitrary",)),
    )(be, xg, sw, w13, w2)


_C_PER_W = _T // 32
_CC = 16


_CNC = _C_PER_W // _CC


@functools.cache
def _make_sc_combine():
    mesh = plsc.VectorSubcoreMesh(core_axis_name="c", subcore_axis_name="s")
    return functools.partial(
        pl.kernel,
        mesh=mesh,
        out_type=jax.ShapeDtypeStruct((_T, _H), jnp.float32),
        scratch_types=[
            [pltpu.VMEM((_CC,), jnp.int32) for _ in range(_CNC)],
            [pltpu.VMEM((_CC,), jnp.int32) for _ in range(_CNC)],
            [pltpu.VMEM((_CC, _H), jnp.float32) for _ in range(2)],
            [pltpu.VMEM((_CC, _H), jnp.float32) for _ in range(2)],
            [pltpu.VMEM((_CC, _H), jnp.float32) for _ in range(2)],
            pltpu.SemaphoreType.DMA,
            [pltpu.SemaphoreType.DMA for _ in range(2)],
            [pltpu.SemaphoreType.DMA for _ in range(2)],
        ],
        compiler_params=pltpu.CompilerParams(needs_layout_passes=False),
    )(_sc_combine_body)


def _sc_combine_body(y_hbm, ys_hbm, d0_hbm, d1_hbm, out_hbm,
                     i0_vs, i1_vs, b0_vs, b1_vs, bs_vs, semi, semg, semw):
    wid = lax.axis_index("s") * 2 + lax.axis_index("c")
    base = wid * _C_PER_W

    cps = [pltpu.async_copy(d0_hbm.at[pl.ds(base + c * _CC, _CC)],
                            i0_vs[c], semi) for c in range(_CNC)]
    cps += [pltpu.async_copy(d1_hbm.at[pl.ds(base + c * _CC, _CC)],
                             i1_vs[c], semi) for c in range(_CNC)]
    for cp in cps:
        cp.wait()

    gcp = [None, None]
    wcp = [None, None]

    def start_gathers(c):
        k = c % 2
        gcp[k] = [
            pltpu.async_copy(y_hbm.at[i0_vs[c]], b0_vs[k], semg[k]),
            pltpu.async_copy(y_hbm.at[i1_vs[c]], b1_vs[k], semg[k]),
            pltpu.async_copy(ys_hbm.at[pl.ds(base + c * _CC, _CC)],
                             bs_vs[k], semg[k]),
        ]

    start_gathers(0)
    for c in range(_CNC):
        k = c % 2
        for cp in gcp[k]:
            cp.wait()
        if c + 1 < _CNC:
            if c >= 1:
                wcp[(c + 1) % 2].wait()
            start_gathers(c + 1)

        def row(r, cr):
            def grp(g, cg):
                sl = pl.ds(g * 16, 16)
                b0_vs[k][r, sl] = (b0_vs[k][r, sl] + b1_vs[k][r, sl]
                                   + bs_vs[k][r, sl])
                return cg
            lax.fori_loop(0, _H // 16, grp, 0)
            return cr

        lax.fori_loop(0, _CC, row, 0)
        wcp[k] = pltpu.async_copy(
            b0_vs[k], out_hbm.at[pl.ds(base + c * _CC, _CC)], semw[k])
    wcp[0].wait()
    wcp[1].wait()


@jax.jit
def kernel(hidden_states, gate_w, e_bias, w13, w2, shared_w13, shared_w2):
    d0, d1, w0, w1, be = _run_router(hidden_states, gate_w, e_bias)
    sw = _make_sc_scatter()(d0.reshape(_T), d1.reshape(_T),
                            w0.reshape(_T), w1.reshape(_T))
    xg = _make_sc_dispatch()(hidden_states, d0.reshape(_T), d1.reshape(_T))
    ys = _run_shared(hidden_states, shared_w13, shared_w2)
    y = _run_gmm(be.reshape(_NB), xg, sw.reshape(_MR, 1), w13, w2)
    return _make_sc_combine()(y, ys, d0.reshape(_T), d1.reshape(_T))


# speedup vs baseline: 1.9936x; 1.0051x over previous
import functools

import jax
import jax.numpy as jnp
from jax import lax
from jax.experimental import pallas as pl
from jax.experimental.pallas import tpu as pltpu
from jax.experimental.pallas import tpu_sc as plsc

_T = 2048
_H = 1024
_E = 16
_I = 512
_NG = 2
_GS = _E // _NG
_RSF = 2.5
_NEG = -1e30

_BT = 256
_NBR = 32
_MR = _NBR * _BT
_NB = _NBR
_SB = 256


_RC = 256
_PC = 128


def _route_chunk(x, gate_w, e_bias):
    logits = jnp.dot(x, gate_w.T, preferred_element_type=jnp.float32)
    s = jax.nn.sigmoid(logits)
    sb = s + e_bias
    lane = lax.broadcasted_iota(jnp.int32, logits.shape, 1)
    gid = lane // _GS

    def top2(v):
        m1 = jnp.max(v, axis=1, keepdims=True)
        i1 = jnp.min(jnp.where(v == m1, lane, _E + 1), axis=1, keepdims=True)
        v2 = jnp.where(lane == i1, _NEG, v)
        m2 = jnp.max(v2, axis=1, keepdims=True)
        i2 = jnp.min(jnp.where(v2 == m2, lane, _E + 1), axis=1, keepdims=True)
        return m1, i1, m2, i2

    m1a, _, m2a, _ = top2(jnp.where(gid == 0, sb, _NEG))
    m1b, _, m2b, _ = top2(jnp.where(gid == 1, sb, _NEG))
    chosen = jnp.where(m1a + m2a >= m1b + m2b, 0, 1)
    masked = jnp.where(gid == chosen, sb, _NEG)
    _, e1, _, e2 = top2(masked)
    w1 = jnp.sum(jnp.where(lane == e1, s, 0.0), axis=1, keepdims=True)
    w2 = jnp.sum(jnp.where(lane == e2, s, 0.0), axis=1, keepdims=True)
    wn = w1 + w2 + 1e-20
    member = ((lane == e1) | (lane == e2)).astype(jnp.float32)
    return e1, e2, (w1 / wn) * _RSF, (w2 / wn) * _RSF, member


def _router_body(x_ref, gate_ref, bias_ref,
                 d0_ref, d1_ref, w0_ref, w1_ref, be_ref, m_ref):
    def step_a(i, carry):
        sl = pl.ds(i * _RC, _RC)
        e1, e2, w1, w2, member = _route_chunk(x_ref[sl, :], gate_ref[...],
                                              bias_ref[...])
        d0_ref[sl] = e1
        d1_ref[sl] = e2
        w0_ref[sl] = w1
        w1_ref[sl] = w2
        m_ref[sl, :] = member
        return carry

    lax.fori_loop(0, _T // _RC, step_a, 0)

    rr = lax.broadcasted_iota(jnp.int32, (_PC, _PC), 0)
    cc = lax.broadcasted_iota(jnp.int32, (_PC, _PC), 1)
    tril = (rr > cc).astype(jnp.float32)

    def step_b(i, carry):
        sl = pl.ds(i * _PC, _PC)
        mc = m_ref[sl, :]
        pos = jnp.dot(tril, mc, preferred_element_type=jnp.float32) + carry
        m_ref[sl, :] = pos
        return carry + jnp.sum(mc, axis=0, keepdims=True)

    counts = lax.fori_loop(0, _T // _PC, step_b,
                           jnp.zeros((1, _E), jnp.float32))

    pc = jnp.ceil(counts / _BT) * _BT
    uu = (lax.broadcasted_iota(jnp.int32, (_E, _E), 0)
          < lax.broadcasted_iota(jnp.int32, (_E, _E), 1)).astype(jnp.float32)
    offs = jnp.dot(pc, uu, preferred_element_type=jnp.float32)
    ends = offs + pc
    bstart = (lax.broadcasted_iota(jnp.int32, (_NB, _E), 0) * _BT
              ).astype(jnp.float32)
    be = jnp.sum((ends <= bstart).astype(jnp.int32), axis=1, keepdims=True)
    be_ref[...] = be

    lane = lax.broadcasted_iota(jnp.int32, (_RC, _E), 1)

    def step_d(i, carry):
        sl = pl.ds(i * _RC, _RC)
        slots = offs + m_ref[sl, :]
        d0_ref[sl] = jnp.sum(jnp.where(lane == d0_ref[sl], slots, 0.0),
                             axis=1, keepdims=True).astype(jnp.int32)
        d1_ref[sl] = jnp.sum(jnp.where(lane == d1_ref[sl], slots, 0.0),
                             axis=1, keepdims=True).astype(jnp.int32)
        return carry

    lax.fori_loop(0, _T // _RC, step_d, 0)


def _run_router(x, gate_w, e_bias):
    grid_spec = pltpu.PrefetchScalarGridSpec(
        num_scalar_prefetch=0,
        grid=(1,),
        in_specs=[
            pl.BlockSpec((_T, _H), lambda i: (0, 0)),
            pl.BlockSpec((_E, _H), lambda i: (0, 0)),
            pl.BlockSpec((1, _E), lambda i: (0, 0)),
        ],
        out_specs=[
            pl.BlockSpec((_T, 1), lambda i: (0, 0)),
            pl.BlockSpec((_T, 1), lambda i: (0, 0)),
            pl.BlockSpec((_T, 1), lambda i: (0, 0)),
            pl.BlockSpec((_T, 1), lambda i: (0, 0)),
            pl.BlockSpec((_NB, 1), lambda i: (0, 0)),
        ],
        scratch_shapes=[pltpu.VMEM((_T, _E), jnp.float32)],
    )
    return pl.pallas_call(
        _router_body,
        grid_spec=grid_spec,
        out_shape=[
            jax.ShapeDtypeStruct((_T, 1), jnp.int32),
            jax.ShapeDtypeStruct((_T, 1), jnp.int32),
            jax.ShapeDtypeStruct((_T, 1), jnp.float32),
            jax.ShapeDtypeStruct((_T, 1), jnp.float32),
            jax.ShapeDtypeStruct((_NB, 1), jnp.int32),
        ],
        compiler_params=pltpu.CompilerParams(
            dimension_semantics=("arbitrary",)),
    )(x, gate_w, e_bias.reshape(1, _E))


_D_PER_W = _T // 32
_DC = 16
_DNC = _D_PER_W // _DC


@functools.cache
def _make_sc_dispatch():
    mesh = plsc.VectorSubcoreMesh(core_axis_name="c", subcore_axis_name="s")
    return functools.partial(
        pl.kernel,
        mesh=mesh,
        out_type=[jax.ShapeDtypeStruct((_MR, _H), jnp.float32),
                  jax.ShapeDtypeStruct((_MR,), jnp.float32)],
        scratch_types=[
            [pltpu.VMEM((_DC,), jnp.int32) for _ in range(_DNC)],
            [pltpu.VMEM((_DC,), jnp.int32) for _ in range(_DNC)],
            [pltpu.VMEM((_DC, _H), jnp.float32) for _ in range(2)],
            pltpu.VMEM((_MR,), jnp.float32),
            pltpu.VMEM((_T,), jnp.int32),
            pltpu.VMEM((_T,), jnp.int32),
            pltpu.VMEM((_T,), jnp.float32),
            pltpu.VMEM((_T,), jnp.float32),
            pltpu.SemaphoreType.DMA,
            [pltpu.SemaphoreType.DMA for _ in range(2)],
            [pltpu.SemaphoreType.DMA for _ in range(2)],
        ],
        compiler_params=pltpu.CompilerParams(needs_layout_passes=False),
    )(_sc_dispatch_body)


def _sc_dispatch_body(x_hbm, d0_hbm, d1_hbm, w0_hbm, w1_hbm, xg_hbm, sw_hbm,
                      i0_vs, i1_vs, xbufs, swt_v, d0_v, d1_v, w0_v, w1_v,
                      semi, semx, semw):
    wid = lax.axis_index("s") * 2 + lax.axis_index("c")
    base = wid * _D_PER_W

    @pl.when(wid == 0)
    def _build_sw():
        acp = [pltpu.async_copy(d0_hbm, d0_v, semi),
               pltpu.async_copy(d1_hbm, d1_v, semi),
               pltpu.async_copy(w0_hbm, w0_v, semi),
               pltpu.async_copy(w1_hbm, w1_v, semi)]
        ones_f = jnp.full((16,), 1.0, jnp.float32)

        def init(i, c):
            swt_v[pl.ds(i * 16, 16)] = ones_f
            return c

        lax.fori_loop(0, _MR // 16, init, 0)
        for cp in acp:
            cp.wait()

        def scat(i, c):
            sl = pl.ds(i * 16, 16)
            plsc.store_scatter(swt_v, [d0_v[sl]], w0_v[sl])
            plsc.store_scatter(swt_v, [d1_v[sl]], w1_v[sl])
            return c

        lax.fori_loop(0, _T // 16, scat, 0)
        pltpu.sync_copy(swt_v, sw_hbm)

    cps = [pltpu.async_copy(d0_hbm.at[pl.ds(base + c * _DC, _DC)],
                            i0_vs[c], semi) for c in range(_DNC)]
    cps += [pltpu.async_copy(d1_hbm.at[pl.ds(base + c * _DC, _DC)],
                             i1_vs[c], semi) for c in range(_DNC)]
    for cp in cps:
        cp.wait()

    lcp = [None, None]
    wcp = [None, None]

    def start_load(c):
        lcp[c % 2] = pltpu.async_copy(
            x_hbm.at[pl.ds(base + c * _DC, _DC)], xbufs[c % 2], semx[c % 2])

    start_load(0)
    for c in range(_DNC):
        k = c % 2
        lcp[k].wait()
        if c + 1 < _DNC:
            if c >= 1:
                for cp in wcp[(c + 1) % 2]:
                    cp.wait()
            start_load(c + 1)
        wcp[k] = [
            pltpu.async_copy(xbufs[k], xg_hbm.at[i0_vs[c]], semw[k]),
            pltpu.async_copy(xbufs[k], xg_hbm.at[i1_vs[c]], semw[k]),
        ]
    for cp in wcp[0] + wcp[1]:
        cp.wait()


def _gmm_body(be_ref, xg_ref, sw_ref, w13_ref, w2_ref, y_ref):
    b = pl.program_id(0)
    be = be_ref[b]

    @pl.when(be < _E)
    def _routed():
        gu = jnp.dot(xg_ref[...], w13_ref[0].T,
                     preferred_element_type=jnp.float32)
        act = jax.nn.silu(gu[:, :_I]) * gu[:, _I:]
        act = act * sw_ref[...]
        y_ref[...] = jnp.dot(act, w2_ref[0].T,
                             preferred_element_type=jnp.float32)


def _shared_body(x_ref, sw13_ref, sw2_ref, ys_ref):
    sgu = jnp.dot(x_ref[...], sw13_ref[...].T,
                  preferred_element_type=jnp.float32)
    sact = jax.nn.silu(sgu[:, :_I]) * sgu[:, _I:]
    ys_ref[...] = jnp.dot(sact, sw2_ref[...].T,
                          preferred_element_type=jnp.float32)


def _run_shared(x, shared_w13, shared_w2):
    return pl.pallas_call(
        _shared_body,
        grid=(_T // _SB,),
        in_specs=[
            pl.BlockSpec((_SB, _H), lambda i: (i, 0)),
            pl.BlockSpec((2 * _I, _H), lambda i: (0, 0)),
            pl.BlockSpec((_H, _I), lambda i: (0, 0)),
        ],
        out_specs=pl.BlockSpec((_SB, _H), lambda i: (i, 0)),
        out_shape=jax.ShapeDtypeStruct((_T, _H), jnp.float32),
        compiler_params=pltpu.CompilerParams(
            dimension_semantics=("arbitrary",)),
    )(x, shared_w13, shared_w2)


def _run_gmm(be, xg, sw, w13, w2):
    grid_spec = pltpu.PrefetchScalarGridSpec(
        num_scalar_prefetch=1,
        grid=(_NB,),
        in_specs=[
            pl.BlockSpec((_BT, _H), lambda b, be: (b, 0)),
            pl.BlockSpec((_BT, 1), lambda b, be: (b, 0)),
            pl.BlockSpec((1, 2 * _I, _H),
                         lambda b, be: (jnp.minimum(be[b], _E - 1), 0, 0)),
            pl.BlockSpec((1, _H, _I),
                         lambda b, be: (jnp.minimum(be[b], _E - 1), 0, 0)),
        ],
        out_specs=pl.BlockSpec((_BT, _H), lambda b, be: (b, 0)),
    )
    return pl.pallas_call(
        _gmm_body,
        grid_spec=grid_spec,
        out_shape=jax.ShapeDtypeStruct((_MR, _H), jnp.float32),
        compiler_params=pltpu.CompilerParams(
            dimension_semantics=("arbitrary",)),
    )(be, xg, sw, w13, w2)


_C_PER_W = _T // 32
_CC = 16


_CNC = _C_PER_W // _CC


@functools.cache
def _make_sc_combine():
    mesh = plsc.VectorSubcoreMesh(core_axis_name="c", subcore_axis_name="s")
    return functools.partial(
        pl.kernel,
        mesh=mesh,
        out_type=jax.ShapeDtypeStruct((_T, _H), jnp.float32),
        scratch_types=[
            [pltpu.VMEM((_CC,), jnp.int32) for _ in range(_CNC)],
            [pltpu.VMEM((_CC,), jnp.int32) for _ in range(_CNC)],
            [pltpu.VMEM((_CC, _H), jnp.float32) for _ in range(2)],
            [pltpu.VMEM((_CC, _H), jnp.float32) for _ in range(2)],
            [pltpu.VMEM((_CC, _H), jnp.float32) for _ in range(2)],
            pltpu.SemaphoreType.DMA,
            [pltpu.SemaphoreType.DMA for _ in range(2)],
            [pltpu.SemaphoreType.DMA for _ in range(2)],
        ],
        compiler_params=pltpu.CompilerParams(needs_layout_passes=False),
    )(_sc_combine_body)


def _sc_combine_body(y_hbm, ys_hbm, d0_hbm, d1_hbm, out_hbm,
                     i0_vs, i1_vs, b0_vs, b1_vs, bs_vs, semi, semg, semw):
    wid = lax.axis_index("s") * 2 + lax.axis_index("c")
    base = wid * _C_PER_W

    cps = [pltpu.async_copy(d0_hbm.at[pl.ds(base + c * _CC, _CC)],
                            i0_vs[c], semi) for c in range(_CNC)]
    cps += [pltpu.async_copy(d1_hbm.at[pl.ds(base + c * _CC, _CC)],
                             i1_vs[c], semi) for c in range(_CNC)]
    for cp in cps:
        cp.wait()

    gcp = [None, None]
    wcp = [None, None]

    def start_gathers(c):
        k = c % 2
        gcp[k] = [
            pltpu.async_copy(y_hbm.at[i0_vs[c]], b0_vs[k], semg[k]),
            pltpu.async_copy(y_hbm.at[i1_vs[c]], b1_vs[k], semg[k]),
            pltpu.async_copy(ys_hbm.at[pl.ds(base + c * _CC, _CC)],
                             bs_vs[k], semg[k]),
        ]

    start_gathers(0)
    for c in range(_CNC):
        k = c % 2
        for cp in gcp[k]:
            cp.wait()
        if c + 1 < _CNC:
            if c >= 1:
                wcp[(c + 1) % 2].wait()
            start_gathers(c + 1)

        def row(r, cr):
            def grp(g, cg):
                sl = pl.ds(g * 16, 16)
                b0_vs[k][r, sl] = (b0_vs[k][r, sl] + b1_vs[k][r, sl]
                                   + bs_vs[k][r, sl])
                return cg
            lax.fori_loop(0, _H // 16, grp, 0)
            return cr

        lax.fori_loop(0, _CC, row, 0)
        wcp[k] = pltpu.async_copy(
            b0_vs[k], out_hbm.at[pl.ds(base + c * _CC, _CC)], semw[k])
    wcp[0].wait()
    wcp[1].wait()


@jax.jit
def kernel(hidden_states, gate_w, e_bias, w13, w2, shared_w13, shared_w2):
    d0, d1, w0, w1, be = _run_router(hidden_states, gate_w, e_bias)
    xg, sw = _make_sc_dispatch()(hidden_states, d0.reshape(_T), d1.reshape(_T),
                                 w0.reshape(_T), w1.reshape(_T))
    ys = _run_shared(hidden_states, shared_w13, shared_w2)
    y = _run_gmm(be.reshape(_NB), xg, sw.reshape(_MR, 1), w13, w2)
    return _make_sc_combine()(y, ys, d0.reshape(_T), d1.reshape(_T))
